# baseline (speedup 1.0000x reference)
import functools

import jax
import jax.numpy as jnp
from jax import lax
from jax.experimental import pallas as pl
from jax.experimental.pallas import tpu as pltpu
from jax.experimental.pallas import tpu_sc as plsc

IN_F = 4096
OUT_F = 4096
GS = 8
GROUPS = IN_F // GS
NCB = 2
CB_SIZE = 2 ** 16
IDX_PER_ROW = GROUPS * NCB
NCHUNK = IDX_PER_ROW // 128
NC, NS = 2, 16
NW = NC * NS
ROWS_PER_W = OUT_F // NW


def _sc_gather_body(table_hbm, idx_hbm, wx_hbm, idx_v, rows_v, sem):
    wid = lax.axis_index("s") * NC + lax.axis_index("c")
    lane = lax.iota(jnp.int32, 16)
    cb_off = (lane & 1) << 16

    def row_body(i, carry):
        o = wid * ROWS_PER_W + i
        pltpu.sync_copy(idx_hbm.at[o], idx_v)
        for k in range(NCHUNK):
            for j in range(8):
                idx_v[k, pl.ds(j * 16, 16)] = (
                    idx_v[k, pl.ds(j * 16, 16)] + cb_off
                )
        copies = [
            pltpu.async_copy(
                table_hbm.at[idx_v.at[k]],
                rows_v.at[pl.ds(k * 128, 128)],
                sem,
            )
            for k in range(NCHUNK)
        ]
        for cp in copies:
            cp.wait()
        pltpu.sync_copy(rows_v, wx_hbm.at[o])
        return carry

    lax.fori_loop(0, ROWS_PER_W, row_body, 0)


@jax.jit
def _sc_gather(table, idx):
    mesh = plsc.VectorSubcoreMesh(core_axis_name="c", subcore_axis_name="s")
    f = functools.partial(
        pl.kernel,
        out_type=jax.ShapeDtypeStruct((OUT_F, IDX_PER_ROW, GS), jnp.float32),
        mesh=mesh,
        scratch_types=[
            pltpu.VMEM((NCHUNK, 128), jnp.int32),
            pltpu.VMEM((IDX_PER_ROW, GS), jnp.float32),
            pltpu.SemaphoreType.DMA,
        ],
        compiler_params=pltpu.CompilerParams(use_tc_tiling_on_sc=False),
    )(_sc_gather_body)
    return f(table, idx)


def _mm_body(x_ref, w_ref, s_ref, b_ref, o_ref):
    acc = lax.dot_general(
        x_ref[...],
        w_ref[...],
        (((1,), (1,)), ((), ())),
        preferred_element_type=jnp.float32,
    )
    o_ref[...] = acc * s_ref[...] + b_ref[...]


@jax.jit
def _tc_matmul(x2, wcat, scales_row, bias_row):
    bn = 256
    grid = OUT_F // bn
    return pl.pallas_call(
        _mm_body,
        grid=(grid,),
        in_specs=[
            pl.BlockSpec((32, NCB * IN_F), lambda j: (0, 0)),
            pl.BlockSpec((bn, NCB * IN_F), lambda j: (j, 0)),
            pl.BlockSpec((1, bn), lambda j: (0, j)),
            pl.BlockSpec((1, bn), lambda j: (0, j)),
        ],
        out_specs=pl.BlockSpec((32, bn), lambda j: (0, j)),
        out_shape=jax.ShapeDtypeStruct((32, OUT_F), jnp.float32),
    )(x2, wcat, scales_row, bias_row)


def kernel(input, codes, codebooks, scales, bias):
    table = codebooks.reshape(NCB * CB_SIZE, GS)
    idx = codes.reshape(OUT_F, NCHUNK, 128)
    wil = _sc_gather(table, idx).reshape(OUT_F, NCB * IN_F)
    xil = jnp.broadcast_to(
        input.reshape(32, GROUPS, 1, GS), (32, GROUPS, NCB, GS)
    ).reshape(32, NCB * IN_F)
    return _tc_matmul(
        xil, wil, scales.reshape(1, OUT_F), bias.reshape(1, OUT_F)
    )


# speedup vs baseline: 3.5947x; 3.5947x over previous
import functools

import jax
import jax.numpy as jnp
from jax import lax
from jax.experimental import pallas as pl
from jax.experimental.pallas import tpu as pltpu
from jax.experimental.pallas import tpu_sc as plsc

IN_F = 4096
OUT_F = 4096
GS = 8
GROUPS = IN_F // GS
NCB = 2
CB_SIZE = 2 ** 16
IDX_PER_ROW = GROUPS * NCB
NCHUNK = IDX_PER_ROW // 128
NC, NS = 2, 16
NW = NC * NS
ROWS_PER_W = OUT_F // NW
RB = OUT_F // 8
KB = IN_F // 128


def _sc_gather_body(table_hbm, idx_hbm, w4_hbm, idx_v, rows_v, wsum_v, sem):
    wid = lax.axis_index("s") * NC + lax.axis_index("c")
    lane = lax.iota(jnp.int32, 16)
    cb_off = (lane & 1) << 16
    half = lane >> 3
    lanemod = lane & 7

    def row_body(i, carry):
        o = wid * ROWS_PER_W + i
        pltpu.sync_copy(idx_hbm.at[o], idx_v)
        for k in range(NCHUNK):
            for j in range(8):
                idx_v[k, pl.ds(j * 16, 16)] = (
                    idx_v[k, pl.ds(j * 16, 16)] + cb_off
                )
        copies = [
            pltpu.async_copy(
                table_hbm.at[idx_v.at[k]],
                rows_v.at[pl.ds(k * 128, 128)],
                sem,
            )
            for k in range(NCHUNK)
        ]
        for cp in copies:
            cp.wait()

        def sum_body(t, carry2):
            base = 4 * t + 2 * half
            a = plsc.load_gather(rows_v, [base, lanemod])
            b = plsc.load_gather(rows_v, [base + 1, lanemod])
            wsum_v[t >> 3, pl.ds((t & 7) * 16, 16)] = a + b
            return carry2

        lax.fori_loop(0, IN_F // 16, sum_body, 0)
        pltpu.sync_copy(wsum_v, w4_hbm.at[o // 8, :, o % 8, :])
        return carry

    lax.fori_loop(0, ROWS_PER_W, row_body, 0)


@jax.jit
def _sc_gather(table, idx):
    mesh = plsc.VectorSubcoreMesh(core_axis_name="c", subcore_axis_name="s")
    f = functools.partial(
        pl.kernel,
        out_type=jax.ShapeDtypeStruct((RB, KB, 8, 128), jnp.float32),
        mesh=mesh,
        scratch_types=[
            pltpu.VMEM((NCHUNK, 128), jnp.int32),
            pltpu.VMEM((IDX_PER_ROW, GS), jnp.float32),
            pltpu.VMEM((KB, 128), jnp.float32),
            pltpu.SemaphoreType.DMA,
        ],
        compiler_params=pltpu.CompilerParams(
            use_tc_tiling_on_sc=False, needs_layout_passes=False
        ),
    )(_sc_gather_body)
    return f(table, idx)


def _mm_body(x_ref, w_ref, s_ref, b_ref, o_ref):
    k = pl.program_id(1)
    w = w_ref[...]
    w2 = w.reshape(w.shape[0] * 8, 128)
    acc = lax.dot_general(
        x_ref[...],
        w2,
        (((1,), (1,)), ((), ())),
        preferred_element_type=jnp.float32,
    )
    prev = jnp.where(k == 0, jnp.zeros_like(acc), o_ref[...])
    tot = prev + acc

    @pl.when(k < KB - 1)
    def _():
        o_ref[...] = tot

    @pl.when(k == KB - 1)
    def _():
        o_ref[...] = tot * s_ref[...] + b_ref[...]


@jax.jit
def _tc_matmul(x, w4, scales_row, bias_row):
    rbb = 64
    grid = (RB // rbb, KB)
    return pl.pallas_call(
        _mm_body,
        grid=grid,
        in_specs=[
            pl.BlockSpec((32, 128), lambda j, k: (0, k)),
            pl.BlockSpec((rbb, 1, 8, 128), lambda j, k: (j, k, 0, 0)),
            pl.BlockSpec((1, rbb * 8), lambda j, k: (0, j)),
            pl.BlockSpec((1, rbb * 8), lambda j, k: (0, j)),
        ],
        out_specs=pl.BlockSpec((32, rbb * 8), lambda j, k: (0, j)),
        out_shape=jax.ShapeDtypeStruct((32, OUT_F), jnp.float32),
        compiler_params=pltpu.CompilerParams(
            dimension_semantics=("arbitrary", "arbitrary")
        ),
    )(x, w4, scales_row, bias_row)


def kernel(input, codes, codebooks, scales, bias):
    table = codebooks.reshape(NCB * CB_SIZE, GS)
    idx = codes.reshape(OUT_F, NCHUNK, 128)
    w4 = _sc_gather(table, idx)
    return _tc_matmul(
        input, w4, scales.reshape(1, OUT_F), bias.reshape(1, OUT_F)
    )


# speedup vs baseline: 3.9549x; 1.1002x over previous
import functools

import jax
import jax.numpy as jnp
from jax import lax
from jax.experimental import pallas as pl
from jax.experimental.pallas import tpu as pltpu
from jax.experimental.pallas import tpu_sc as plsc

IN_F = 4096
OUT_F = 4096
GS = 8
GROUPS = IN_F // GS
NCB = 2
CB_SIZE = 2 ** 16
IDX_PER_ROW = GROUPS * NCB
NCHUNK = IDX_PER_ROW // 128
NC, NS = 2, 16
NW = NC * NS
ROWS_PER_W = OUT_F // NW
RB = OUT_F // 8
KB = IN_F // 128


def _sc_gather_body(table_hbm, idx_hbm, w4_hbm, idx_v, rows_v, wsum_v, sem):
    wid = lax.axis_index("s") * NC + lax.axis_index("c")
    lane = lax.iota(jnp.int32, 16)
    cb_off = (lane & 1) << 16
    half = lane >> 3
    lanemod = lane & 7

    def row_body(i, carry):
        o = wid * ROWS_PER_W + i
        pltpu.sync_copy(idx_hbm.at[o], idx_v)

        def off_body(j, carry3):
            idx_v[pl.ds(j * 16, 16)] = idx_v[pl.ds(j * 16, 16)] + cb_off
            return carry3

        lax.fori_loop(0, IDX_PER_ROW // 16, off_body, 0)
        pltpu.async_copy(table_hbm.at[idx_v], rows_v, sem).wait()

        def sum_body(t, carry2):
            base = 4 * t + 2 * half
            a = plsc.load_gather(rows_v, [base, lanemod])
            b = plsc.load_gather(rows_v, [base + 1, lanemod])
            wsum_v[t >> 3, pl.ds((t & 7) * 16, 16)] = a + b
            return carry2

        lax.fori_loop(0, IN_F // 16, sum_body, 0)
        pltpu.sync_copy(wsum_v, w4_hbm.at[o // 8, :, o % 8, :])
        return carry

    lax.fori_loop(0, ROWS_PER_W, row_body, 0)


@jax.jit
def _sc_gather(table, idx):
    mesh = plsc.VectorSubcoreMesh(core_axis_name="c", subcore_axis_name="s")
    f = functools.partial(
        pl.kernel,
        out_type=jax.ShapeDtypeStruct((RB, KB, 8, 128), jnp.float32),
        mesh=mesh,
        scratch_types=[
            pltpu.VMEM((IDX_PER_ROW,), jnp.int32),
            pltpu.VMEM((IDX_PER_ROW, GS), jnp.float32),
            pltpu.VMEM((KB, 128), jnp.float32),
            pltpu.SemaphoreType.DMA,
        ],
        compiler_params=pltpu.CompilerParams(
            use_tc_tiling_on_sc=False, needs_layout_passes=False
        ),
    )(_sc_gather_body)
    return f(table, idx)


def _mm_body(x_ref, w_ref, s_ref, b_ref, o_ref):
    acc = jnp.zeros((32, w_ref.shape[0] * 8), jnp.float32)
    for kb in range(KB):
        w2 = w_ref[:, kb].reshape(w_ref.shape[0] * 8, 128)
        acc = acc + lax.dot_general(
            x_ref[:, pl.ds(kb * 128, 128)],
            w2,
            (((1,), (1,)), ((), ())),
            preferred_element_type=jnp.float32,
        )
    o_ref[...] = acc * s_ref[...] + b_ref[...]


@jax.jit
def _tc_matmul(x, w4, scales_row, bias_row):
    rbb = 64
    grid = (RB // rbb,)
    return pl.pallas_call(
        _mm_body,
        grid=grid,
        in_specs=[
            pl.BlockSpec((32, IN_F), lambda j: (0, 0)),
            pl.BlockSpec((rbb, KB, 8, 128), lambda j: (j, 0, 0, 0)),
            pl.BlockSpec((1, rbb * 8), lambda j: (0, j)),
            pl.BlockSpec((1, rbb * 8), lambda j: (0, j)),
        ],
        out_specs=pl.BlockSpec((32, rbb * 8), lambda j: (0, j)),
        out_shape=jax.ShapeDtypeStruct((32, OUT_F), jnp.float32),
        compiler_params=pltpu.CompilerParams(
            dimension_semantics=("arbitrary",)
        ),
    )(x, w4, scales_row, bias_row)


def kernel(input, codes, codebooks, scales, bias):
    table = codebooks.reshape(NCB * CB_SIZE, GS)
    idx = codes.reshape(OUT_F, IDX_PER_ROW)
    w4 = _sc_gather(table, idx)
    return _tc_matmul(
        input, w4, scales.reshape(1, OUT_F), bias.reshape(1, OUT_F)
    )


# speedup vs baseline: 6.6403x; 1.6790x over previous
import functools

import jax
import jax.numpy as jnp
from jax import lax
from jax.experimental import pallas as pl
from jax.experimental.pallas import tpu as pltpu
from jax.experimental.pallas import tpu_sc as plsc

IN_F = 4096
OUT_F = 4096
GS = 8
GROUPS = IN_F // GS
NCB = 2
CB_SIZE = 2 ** 16
IDX_PER_ROW = GROUPS * NCB
NCHUNK = IDX_PER_ROW // 128
NC, NS = 2, 16
NW = NC * NS
ROWS_PER_W = OUT_F // NW
RB = OUT_F // 8
KB = IN_F // 128


def _sc_gather_body(
    table_hbm, idx_hbm, w4_hbm, idxb, rows, ws, isem, gsem, wsem
):
    wid = lax.axis_index("s") * NC + lax.axis_index("c")
    base_row = wid * ROWS_PER_W
    lane = lax.iota(jnp.int32, 16)
    cb_off = (lane & 1) << 16
    half = lane >> 3
    lanemod = lane & 7

    def add_offsets(slot):
        def off_body(j, carry3):
            idxb[slot, pl.ds(j * 16, 16)] = (
                idxb[slot, pl.ds(j * 16, 16)] + cb_off
            )
            return carry3

        lax.fori_loop(0, IDX_PER_ROW // 16, off_body, 0, unroll=8)

    def fire_gather(slot):
        pltpu.async_copy(table_hbm.at[idxb.at[slot]], rows.at[slot], gsem)

    pltpu.sync_copy(idx_hbm.at[base_row], idxb.at[0])
    add_offsets(0)
    fire_gather(0)
    pltpu.async_copy(idx_hbm.at[base_row + 1], idxb.at[1], isem)

    def row_body(r, carry):
        o = base_row + r
        p = r & 1
        q = 1 - p
        pltpu.make_async_copy(table_hbm.at[idxb.at[p]], rows.at[p], gsem).wait()

        @pl.when(r + 2 < ROWS_PER_W)
        def _():
            pltpu.async_copy(idx_hbm.at[o + 2], idxb.at[p], isem)

        @pl.when(r + 1 < ROWS_PER_W)
        def _():
            pltpu.make_async_copy(
                idx_hbm.at[o + 1], idxb.at[q], isem
            ).wait()
            add_offsets(q)
            fire_gather(q)

        @pl.when(r >= 2)
        def _():
            pltpu.make_async_copy(
                ws.at[p], w4_hbm.at[(o - 2) // 8, :, (o - 2) % 8, :], wsem
            ).wait()

        pvec = lane * 0 + p

        def sum_body(t, carry2):
            base = 4 * t + 2 * half
            a = plsc.load_gather(rows, [pvec, base, lanemod])
            b = plsc.load_gather(rows, [pvec, base + 1, lanemod])
            ws[p, t >> 3, pl.ds((t & 7) * 16, 16)] = a + b
            return carry2

        lax.fori_loop(0, IN_F // 16, sum_body, 0, unroll=8)
        pltpu.async_copy(ws.at[p], w4_hbm.at[o // 8, :, o % 8, :], wsem)
        return carry

    lax.fori_loop(0, ROWS_PER_W, row_body, 0)

    last = base_row + ROWS_PER_W - 2
    pltpu.make_async_copy(
        ws.at[0], w4_hbm.at[last // 8, :, last % 8, :], wsem
    ).wait()
    pltpu.make_async_copy(
        ws.at[1], w4_hbm.at[(last + 1) // 8, :, (last + 1) % 8, :], wsem
    ).wait()


@jax.jit
def _sc_gather(table, idx):
    mesh = plsc.VectorSubcoreMesh(core_axis_name="c", subcore_axis_name="s")
    f = functools.partial(
        pl.kernel,
        out_type=jax.ShapeDtypeStruct((RB, KB, 8, 128), jnp.float32),
        mesh=mesh,
        scratch_types=[
            pltpu.VMEM((2, IDX_PER_ROW), jnp.int32),
            pltpu.VMEM((2, IDX_PER_ROW, GS), jnp.float32),
            pltpu.VMEM((2, KB, 128), jnp.float32),
            pltpu.SemaphoreType.DMA,
            pltpu.SemaphoreType.DMA,
            pltpu.SemaphoreType.DMA,
        ],
        compiler_params=pltpu.CompilerParams(
            use_tc_tiling_on_sc=False, needs_layout_passes=False
        ),
    )(_sc_gather_body)
    return f(table, idx)


def _mm_body(x_ref, w_ref, s_ref, b_ref, o_ref):
    acc = jnp.zeros((32, w_ref.shape[0] * 8), jnp.float32)
    for kb in range(KB):
        w2 = w_ref[:, kb].reshape(w_ref.shape[0] * 8, 128)
        acc = acc + lax.dot_general(
            x_ref[:, pl.ds(kb * 128, 128)],
            w2,
            (((1,), (1,)), ((), ())),
            preferred_element_type=jnp.float32,
        )
    o_ref[...] = acc * s_ref[...] + b_ref[...]


@jax.jit
def _tc_matmul(x, w4, scales_row, bias_row):
    rbb = 64
    grid = (RB // rbb,)
    return pl.pallas_call(
        _mm_body,
        grid=grid,
        in_specs=[
            pl.BlockSpec((32, IN_F), lambda j: (0, 0)),
            pl.BlockSpec((rbb, KB, 8, 128), lambda j: (j, 0, 0, 0)),
            pl.BlockSpec((1, rbb * 8), lambda j: (0, j)),
            pl.BlockSpec((1, rbb * 8), lambda j: (0, j)),
        ],
        out_specs=pl.BlockSpec((32, rbb * 8), lambda j: (0, j)),
        out_shape=jax.ShapeDtypeStruct((32, OUT_F), jnp.float32),
        compiler_params=pltpu.CompilerParams(
            dimension_semantics=("arbitrary",)
        ),
    )(x, w4, scales_row, bias_row)


def kernel(input, codes, codebooks, scales, bias):
    table = codebooks.reshape(NCB * CB_SIZE, GS)
    idx = codes.reshape(OUT_F, IDX_PER_ROW)
    w4 = _sc_gather(table, idx)
    return _tc_matmul(
        input, w4, scales.reshape(1, OUT_F), bias.reshape(1, OUT_F)
    )


# speedup vs baseline: 7.0167x; 1.0567x over previous
import functools

import jax
import jax.numpy as jnp
from jax import lax
from jax.experimental import pallas as pl
from jax.experimental.pallas import tpu as pltpu
from jax.experimental.pallas import tpu_sc as plsc

IN_F = 4096
OUT_F = 4096
GS = 8
GROUPS = IN_F // GS
NCB = 2
CB_SIZE = 2 ** 16
IDX_PER_ROW = GROUPS * NCB
NCHUNK = IDX_PER_ROW // 128
NC, NS = 2, 16
NW = NC * NS
ROWS_PER_W = OUT_F // NW
RB = OUT_F // 8
KB = IN_F // 128


def _sc_gather_body(
    table_hbm, idx_hbm, w4_hbm, idxb, rows, ws, isem, gsem, wsem
):
    wid = lax.axis_index("s") * NC + lax.axis_index("c")
    base_row = wid * ROWS_PER_W
    lane = lax.iota(jnp.int32, 16)
    cb_off = (lane & 1) << 16
    half = lane >> 3
    lanemod = lane & 7

    def add_offsets(slot):
        def off_body(j, carry3):
            idxb[slot, pl.ds(j * 16, 16)] = (
                idxb[slot, pl.ds(j * 16, 16)] + cb_off
            )
            return carry3

        lax.fori_loop(0, IDX_PER_ROW // 16, off_body, 0, unroll=8)

    def fire_gather(slot):
        pltpu.async_copy(table_hbm.at[idxb.at[slot]], rows.at[slot], gsem)

    pltpu.sync_copy(idx_hbm.at[base_row], idxb.at[0])
    add_offsets(0)
    fire_gather(0)
    pltpu.sync_copy(idx_hbm.at[base_row + 1], idxb.at[1])
    add_offsets(1)
    fire_gather(1)
    pltpu.async_copy(idx_hbm.at[base_row + 2], idxb.at[2], isem)

    def row_body(r, carry):
        o = base_row + r
        p = lax.rem(r, 3)
        n2 = lax.rem(r + 2, 3)
        pw = r & 1
        pltpu.make_async_copy(table_hbm.at[idxb.at[p]], rows.at[p], gsem).wait()

        @pl.when(r + 3 < ROWS_PER_W)
        def _():
            pltpu.async_copy(idx_hbm.at[o + 3], idxb.at[p], isem)

        @pl.when(r + 2 < ROWS_PER_W)
        def _():
            pltpu.make_async_copy(
                idx_hbm.at[o + 2], idxb.at[n2], isem
            ).wait()
            add_offsets(n2)
            fire_gather(n2)

        @pl.when(r >= 2)
        def _():
            pltpu.make_async_copy(
                ws.at[pw], w4_hbm.at[(o - 2) // 8, :, (o - 2) % 8, :], wsem
            ).wait()

        pvec = lane * 0 + p

        def sum_body(t, carry2):
            base = 4 * t + 2 * half
            a = plsc.load_gather(rows, [pvec, base, lanemod])
            b = plsc.load_gather(rows, [pvec, base + 1, lanemod])
            ws[pw, t >> 3, pl.ds((t & 7) * 16, 16)] = a + b
            return carry2

        lax.fori_loop(0, IN_F // 16, sum_body, 0, unroll=8)
        pltpu.async_copy(ws.at[pw], w4_hbm.at[o // 8, :, o % 8, :], wsem)
        return carry

    lax.fori_loop(0, ROWS_PER_W, row_body, 0)

    last = base_row + ROWS_PER_W - 2
    pltpu.make_async_copy(
        ws.at[0], w4_hbm.at[last // 8, :, last % 8, :], wsem
    ).wait()
    pltpu.make_async_copy(
        ws.at[1], w4_hbm.at[(last + 1) // 8, :, (last + 1) % 8, :], wsem
    ).wait()


@jax.jit
def _sc_gather(table, idx):
    mesh = plsc.VectorSubcoreMesh(core_axis_name="c", subcore_axis_name="s")
    f = functools.partial(
        pl.kernel,
        out_type=jax.ShapeDtypeStruct((RB, KB, 8, 128), jnp.float32),
        mesh=mesh,
        scratch_types=[
            pltpu.VMEM((3, IDX_PER_ROW), jnp.int32),
            pltpu.VMEM((3, IDX_PER_ROW, GS), jnp.float32),
            pltpu.VMEM((2, KB, 128), jnp.float32),
            pltpu.SemaphoreType.DMA,
            pltpu.SemaphoreType.DMA,
            pltpu.SemaphoreType.DMA,
        ],
        compiler_params=pltpu.CompilerParams(
            use_tc_tiling_on_sc=False, needs_layout_passes=False
        ),
    )(_sc_gather_body)
    return f(table, idx)


def _mm_body(x_ref, w_ref, s_ref, b_ref, o_ref):
    acc = jnp.zeros((32, w_ref.shape[0] * 8), jnp.float32)
    for kb in range(KB):
        w2 = w_ref[:, kb].reshape(w_ref.shape[0] * 8, 128)
        acc = acc + lax.dot_general(
            x_ref[:, pl.ds(kb * 128, 128)],
            w2,
            (((1,), (1,)), ((), ())),
            preferred_element_type=jnp.float32,
        )
    o_ref[...] = acc * s_ref[...] + b_ref[...]


@jax.jit
def _tc_matmul(x, w4, scales_row, bias_row):
    rbb = 64
    grid = (RB // rbb,)
    return pl.pallas_call(
        _mm_body,
        grid=grid,
        in_specs=[
            pl.BlockSpec((32, IN_F), lambda j: (0, 0)),
            pl.BlockSpec((rbb, KB, 8, 128), lambda j: (j, 0, 0, 0)),
            pl.BlockSpec((1, rbb * 8), lambda j: (0, j)),
            pl.BlockSpec((1, rbb * 8), lambda j: (0, j)),
        ],
        out_specs=pl.BlockSpec((32, rbb * 8), lambda j: (0, j)),
        out_shape=jax.ShapeDtypeStruct((32, OUT_F), jnp.float32),
        compiler_params=pltpu.CompilerParams(
            dimension_semantics=("arbitrary",)
        ),
    )(x, w4, scales_row, bias_row)


def kernel(input, codes, codebooks, scales, bias):
    table = codebooks.reshape(NCB * CB_SIZE, GS)
    idx = codes.reshape(OUT_F, IDX_PER_ROW)
    w4 = _sc_gather(table, idx)
    return _tc_matmul(
        input, w4, scales.reshape(1, OUT_F), bias.reshape(1, OUT_F)
    )


# speedup vs baseline: 7.4313x; 1.0591x over previous
import functools

import jax
import jax.numpy as jnp
from jax import lax
from jax.experimental import pallas as pl
from jax.experimental.pallas import tpu as pltpu
from jax.experimental.pallas import tpu_sc as plsc

IN_F = 4096
OUT_F = 4096
GS = 8
GROUPS = IN_F // GS
NCB = 2
CB_SIZE = 2 ** 16
IDX_PER_ROW = GROUPS * NCB
NCHUNK = IDX_PER_ROW // 128
NC, NS = 2, 16
NW = NC * NS
ROWS_PER_W = OUT_F // NW
RB = OUT_F // 8
KB = IN_F // 128


def _sc_gather_body(
    table_hbm, idx_hbm, w4_hbm, idxb, rows, ws, table_sp, isem, gsem, wsem
):
    sid = lax.axis_index("s")
    wid = sid * NC + lax.axis_index("c")
    base_row = wid * ROWS_PER_W

    @pl.when(sid == 0)
    def _():
        pltpu.sync_copy(table_hbm, table_sp)

    plsc.subcore_barrier()
    lane = lax.iota(jnp.int32, 16)
    cb_off = (lane & 1) << 16
    half = lane >> 3
    lanemod = lane & 7

    def add_offsets(slot):
        def off_body(j, carry3):
            idxb[slot, pl.ds(j * 16, 16)] = (
                idxb[slot, pl.ds(j * 16, 16)] + cb_off
            )
            return carry3

        lax.fori_loop(0, IDX_PER_ROW // 16, off_body, 0, unroll=8)

    def fire_gather(slot):
        pltpu.async_copy(table_sp.at[idxb.at[slot]], rows.at[slot], gsem)

    pltpu.sync_copy(idx_hbm.at[base_row], idxb.at[0])
    add_offsets(0)
    fire_gather(0)
    pltpu.sync_copy(idx_hbm.at[base_row + 1], idxb.at[1])
    add_offsets(1)
    fire_gather(1)
    pltpu.async_copy(idx_hbm.at[base_row + 2], idxb.at[2], isem)

    def row_body(r, carry):
        o = base_row + r
        p = lax.rem(r, 3)
        n2 = lax.rem(r + 2, 3)
        pw = r & 1
        pltpu.make_async_copy(table_sp.at[idxb.at[p]], rows.at[p], gsem).wait()

        @pl.when(r + 3 < ROWS_PER_W)
        def _():
            pltpu.async_copy(idx_hbm.at[o + 3], idxb.at[p], isem)

        @pl.when(r + 2 < ROWS_PER_W)
        def _():
            pltpu.make_async_copy(
                idx_hbm.at[o + 2], idxb.at[n2], isem
            ).wait()
            add_offsets(n2)
            fire_gather(n2)

        @pl.when(r >= 2)
        def _():
            pltpu.make_async_copy(
                ws.at[pw], w4_hbm.at[(o - 2) // 8, :, (o - 2) % 8, :], wsem
            ).wait()

        pvec = lane * 0 + p

        def sum_body(t, carry2):
            base = 4 * t + 2 * half
            a = plsc.load_gather(rows, [pvec, base, lanemod])
            b = plsc.load_gather(rows, [pvec, base + 1, lanemod])
            ws[pw, t >> 3, pl.ds((t & 7) * 16, 16)] = a + b
            return carry2

        lax.fori_loop(0, IN_F // 16, sum_body, 0, unroll=8)
        pltpu.async_copy(ws.at[pw], w4_hbm.at[o // 8, :, o % 8, :], wsem)
        return carry

    lax.fori_loop(0, ROWS_PER_W, row_body, 0)

    last = base_row + ROWS_PER_W - 2
    pltpu.make_async_copy(
        ws.at[0], w4_hbm.at[last // 8, :, last % 8, :], wsem
    ).wait()
    pltpu.make_async_copy(
        ws.at[1], w4_hbm.at[(last + 1) // 8, :, (last + 1) % 8, :], wsem
    ).wait()


@jax.jit
def _sc_gather(table, idx):
    mesh = plsc.VectorSubcoreMesh(core_axis_name="c", subcore_axis_name="s")
    f = functools.partial(
        pl.kernel,
        out_type=jax.ShapeDtypeStruct((RB, KB, 8, 128), jnp.float32),
        mesh=mesh,
        scratch_types=[
            pltpu.VMEM((3, IDX_PER_ROW), jnp.int32),
            pltpu.VMEM((3, IDX_PER_ROW, GS), jnp.float32),
            pltpu.VMEM((2, KB, 128), jnp.float32),
            pltpu.VMEM_SHARED((NCB * CB_SIZE, GS), jnp.float32),
            pltpu.SemaphoreType.DMA,
            pltpu.SemaphoreType.DMA,
            pltpu.SemaphoreType.DMA,
        ],
        compiler_params=pltpu.CompilerParams(
            use_tc_tiling_on_sc=False, needs_layout_passes=False
        ),
    )(_sc_gather_body)
    return f(table, idx)


def _mm_body(x_ref, w_ref, s_ref, b_ref, o_ref):
    acc = jnp.zeros((32, w_ref.shape[0] * 8), jnp.float32)
    for kb in range(KB):
        w2 = w_ref[:, kb].reshape(w_ref.shape[0] * 8, 128)
        acc = acc + lax.dot_general(
            x_ref[:, pl.ds(kb * 128, 128)],
            w2,
            (((1,), (1,)), ((), ())),
            preferred_element_type=jnp.float32,
        )
    o_ref[...] = acc * s_ref[...] + b_ref[...]


@jax.jit
def _tc_matmul(x, w4, scales_row, bias_row):
    rbb = 64
    grid = (RB // rbb,)
    return pl.pallas_call(
        _mm_body,
        grid=grid,
        in_specs=[
            pl.BlockSpec((32, IN_F), lambda j: (0, 0)),
            pl.BlockSpec((rbb, KB, 8, 128), lambda j: (j, 0, 0, 0)),
            pl.BlockSpec((1, rbb * 8), lambda j: (0, j)),
            pl.BlockSpec((1, rbb * 8), lambda j: (0, j)),
        ],
        out_specs=pl.BlockSpec((32, rbb * 8), lambda j: (0, j)),
        out_shape=jax.ShapeDtypeStruct((32, OUT_F), jnp.float32),
        compiler_params=pltpu.CompilerParams(
            dimension_semantics=("arbitrary",)
        ),
    )(x, w4, scales_row, bias_row)


def kernel(input, codes, codebooks, scales, bias):
    table = codebooks.reshape(NCB * CB_SIZE, GS)
    idx = codes.reshape(OUT_F, IDX_PER_ROW)
    w4 = _sc_gather(table, idx)
    return _tc_matmul(
        input, w4, scales.reshape(1, OUT_F), bias.reshape(1, OUT_F)
    )


# speedup vs baseline: 7.7455x; 1.0423x over previous
import functools

import jax
import jax.numpy as jnp
from jax import lax
from jax.experimental import pallas as pl
from jax.experimental.pallas import tpu as pltpu
from jax.experimental.pallas import tpu_sc as plsc

IN_F = 4096
OUT_F = 4096
GS = 8
GROUPS = IN_F // GS
NCB = 2
CB_SIZE = 2 ** 16
IDX_PER_ROW = GROUPS * NCB
NCHUNK = IDX_PER_ROW // 128
NC, NS = 2, 16
NW = NC * NS
ROWS_PER_W = OUT_F // NW
RB = OUT_F // 8
KB = IN_F // 128


def _sc_gather_body(
    table_hbm, idx_hbm, w4_hbm, idxb, rows, ws, table_sp, isem, gsem, wsem
):
    sid = lax.axis_index("s")
    wid = sid * NC + lax.axis_index("c")
    base_row = wid * ROWS_PER_W

    @pl.when(sid == 0)
    def _():
        pltpu.sync_copy(table_hbm, table_sp)

    plsc.subcore_barrier()
    lane = lax.iota(jnp.int32, 16)
    cb_off = (lane & 1) << 16
    half = lane >> 3
    lanemod = lane & 7

    def add_offsets(slot):
        def off_body(j, carry3):
            idxb[slot, pl.ds(j * 16, 16)] = (
                idxb[slot, pl.ds(j * 16, 16)] + cb_off
            )
            return carry3

        lax.fori_loop(0, IDX_PER_ROW // 16, off_body, 0, unroll=8)

    def fire_gather(slot):
        pltpu.async_copy(table_sp.at[idxb.at[slot]], rows.at[slot], gsem)

    for r0 in range(3):
        pltpu.sync_copy(idx_hbm.at[base_row + r0], idxb.at[r0])
        add_offsets(r0)
        fire_gather(r0)
    pltpu.async_copy(idx_hbm.at[base_row + 3], idxb.at[3], isem)

    def row_body(r, carry):
        o = base_row + r
        p = r & 3
        n3 = (r + 3) & 3
        pw = r & 1
        pltpu.make_async_copy(table_sp.at[idxb.at[p]], rows.at[p], gsem).wait()

        @pl.when(r + 4 < ROWS_PER_W)
        def _():
            pltpu.async_copy(idx_hbm.at[o + 4], idxb.at[p], isem)

        @pl.when(r + 3 < ROWS_PER_W)
        def _():
            pltpu.make_async_copy(
                idx_hbm.at[o + 3], idxb.at[n3], isem
            ).wait()
            add_offsets(n3)
            fire_gather(n3)

        @pl.when(r >= 2)
        def _():
            pltpu.make_async_copy(
                ws.at[pw], w4_hbm.at[(o - 2) // 8, :, (o - 2) % 8, :], wsem
            ).wait()

        pvec = lane * 0 + p

        def sum_body(t, carry2):
            base = 4 * t + 2 * half
            a = plsc.load_gather(rows, [pvec, base, lanemod])
            b = plsc.load_gather(rows, [pvec, base + 1, lanemod])
            ws[pw, t >> 3, pl.ds((t & 7) * 16, 16)] = a + b
            return carry2

        lax.fori_loop(0, IN_F // 16, sum_body, 0, unroll=8)
        pltpu.async_copy(ws.at[pw], w4_hbm.at[o // 8, :, o % 8, :], wsem)
        return carry

    lax.fori_loop(0, ROWS_PER_W, row_body, 0)

    last = base_row + ROWS_PER_W - 2
    pltpu.make_async_copy(
        ws.at[0], w4_hbm.at[last // 8, :, last % 8, :], wsem
    ).wait()
    pltpu.make_async_copy(
        ws.at[1], w4_hbm.at[(last + 1) // 8, :, (last + 1) % 8, :], wsem
    ).wait()


@jax.jit
def _sc_gather(table, idx):
    mesh = plsc.VectorSubcoreMesh(core_axis_name="c", subcore_axis_name="s")
    f = functools.partial(
        pl.kernel,
        out_type=jax.ShapeDtypeStruct((RB, KB, 8, 128), jnp.float32),
        mesh=mesh,
        scratch_types=[
            pltpu.VMEM((4, IDX_PER_ROW), jnp.int32),
            pltpu.VMEM((4, IDX_PER_ROW, GS), jnp.float32),
            pltpu.VMEM((2, KB, 128), jnp.float32),
            pltpu.VMEM_SHARED((NCB * CB_SIZE, GS), jnp.float32),
            pltpu.SemaphoreType.DMA,
            pltpu.SemaphoreType.DMA,
            pltpu.SemaphoreType.DMA,
        ],
        compiler_params=pltpu.CompilerParams(
            use_tc_tiling_on_sc=False, needs_layout_passes=False
        ),
    )(_sc_gather_body)
    return f(table, idx)


def _mm_body(x_ref, w_ref, s_ref, b_ref, o_ref):
    acc = jnp.zeros((32, w_ref.shape[0] * 8), jnp.float32)
    for kb in range(KB):
        w2 = w_ref[:, kb].reshape(w_ref.shape[0] * 8, 128)
        acc = acc + lax.dot_general(
            x_ref[:, pl.ds(kb * 128, 128)],
            w2,
            (((1,), (1,)), ((), ())),
            preferred_element_type=jnp.float32,
        )
    o_ref[...] = acc * s_ref[...] + b_ref[...]


@jax.jit
def _tc_matmul(x, w4, scales_row, bias_row):
    rbb = 64
    grid = (RB // rbb,)
    return pl.pallas_call(
        _mm_body,
        grid=grid,
        in_specs=[
            pl.BlockSpec((32, IN_F), lambda j: (0, 0)),
            pl.BlockSpec((rbb, KB, 8, 128), lambda j: (j, 0, 0, 0)),
            pl.BlockSpec((1, rbb * 8), lambda j: (0, j)),
            pl.BlockSpec((1, rbb * 8), lambda j: (0, j)),
        ],
        out_specs=pl.BlockSpec((32, rbb * 8), lambda j: (0, j)),
        out_shape=jax.ShapeDtypeStruct((32, OUT_F), jnp.float32),
        compiler_params=pltpu.CompilerParams(
            dimension_semantics=("arbitrary",)
        ),
    )(x, w4, scales_row, bias_row)


def kernel(input, codes, codebooks, scales, bias):
    table = codebooks.reshape(NCB * CB_SIZE, GS)
    idx = codes.reshape(OUT_F, IDX_PER_ROW)
    w4 = _sc_gather(table, idx)
    return _tc_matmul(
        input, w4, scales.reshape(1, OUT_F), bias.reshape(1, OUT_F)
    )
